# Initial kernel scaffold; baseline (speedup 1.0000x reference)
import jax
import jax.numpy as jnp
from jax.experimental import pallas as pl


def kernel(x, edge_index, edge_attr, batch, gene_feat, in_W, in_b, in_g, in_bln, gin_W1, gin_b1, gin_W2, gin_b2, eps, ln_g, ln_b, gp_W1, gp_b1, gp_g, gp_bln, gp_W2, gp_b2, cls_W1, cls_b1, cls_W2, cls_b2, cls_W3, cls_b3):
    raise NotImplementedError("write your pallas kernel here")


# speedup vs baseline: 1.5628x; 1.5628x over previous
import functools
import jax
import jax.numpy as jnp
from jax.experimental import pallas as pl
from jax.experimental.pallas import tpu as pltpu

H = 256
ROWS = 2000


def _ln_rows(y, g, b):
    m = jnp.mean(y, axis=-1, keepdims=True)
    v = jnp.mean((y - m) * (y - m), axis=-1, keepdims=True)
    return (y - m) * jax.lax.rsqrt(v + 1e-5) * g + b


def _in_body(x_ref, w_ref, b_ref, g_ref, bl_ref, o_ref):
    y = jnp.dot(x_ref[...], w_ref[...], preferred_element_type=jnp.float32)
    y = y + b_ref[...]
    y = _ln_rows(y, g_ref[...], bl_ref[...])
    o_ref[...] = jnp.maximum(y, 0.0)


def _input_layer(x, in_W, in_b, in_g, in_bln):
    n = x.shape[0]
    grid = n // ROWS
    return pl.pallas_call(
        _in_body,
        grid=(grid,),
        in_specs=[
            pl.BlockSpec((ROWS, x.shape[1]), lambda i: (i, 0)),
            pl.BlockSpec((x.shape[1], H), lambda i: (0, 0)),
            pl.BlockSpec((1, H), lambda i: (0, 0)),
            pl.BlockSpec((1, H), lambda i: (0, 0)),
            pl.BlockSpec((1, H), lambda i: (0, 0)),
        ],
        out_specs=pl.BlockSpec((ROWS, H), lambda i: (i, 0)),
        out_shape=jax.ShapeDtypeStruct((n, H), jnp.float32),
    )(x, in_W, in_b.reshape(1, H), in_g.reshape(1, H), in_bln.reshape(1, H))


def _gin_body(s_ref, h_ref, a_ref, w1_ref, b1_ref, w2_ref, b2_ref,
              g_ref, bl_ref, o_ref):
    h = h_ref[...]
    t = s_ref[0, 0] * h + a_ref[...]
    u = jnp.dot(t, w1_ref[...], preferred_element_type=jnp.float32)
    u = jnp.maximum(u + b1_ref[...], 0.0)
    y = jnp.dot(u, w2_ref[...], preferred_element_type=jnp.float32)
    y = y + b2_ref[...]
    y = jnp.maximum(_ln_rows(y, g_ref[...], bl_ref[...]), 0.0)
    o_ref[...] = y + h


def _gin_dense(h, agg, scale, W1, b1, W2, b2, lng, lnb):
    n = h.shape[0]
    grid = n // ROWS
    return pl.pallas_call(
        _gin_body,
        grid=(grid,),
        in_specs=[
            pl.BlockSpec(memory_space=pltpu.SMEM),
            pl.BlockSpec((ROWS, H), lambda i: (i, 0)),
            pl.BlockSpec((ROWS, H), lambda i: (i, 0)),
            pl.BlockSpec((H, 2 * H), lambda i: (0, 0)),
            pl.BlockSpec((1, 2 * H), lambda i: (0, 0)),
            pl.BlockSpec((2 * H, H), lambda i: (0, 0)),
            pl.BlockSpec((1, H), lambda i: (0, 0)),
            pl.BlockSpec((1, H), lambda i: (0, 0)),
            pl.BlockSpec((1, H), lambda i: (0, 0)),
        ],
        out_specs=pl.BlockSpec((ROWS, H), lambda i: (i, 0)),
        out_shape=jax.ShapeDtypeStruct((n, H), jnp.float32),
    )(scale.reshape(1, 1), h, agg, W1, b1.reshape(1, 2 * H), W2,
      b2.reshape(1, H), lng.reshape(1, H), lnb.reshape(1, H))


def _pool_body(h_ref, b_ref, gf_ref, gw1_ref, gb1_ref, gg_ref, gbl_ref,
               gw2_ref, gb2_ref, cw1_ref, cb1_ref, cw2_ref, cb2_ref,
               cw3_ref, cb3_ref, o_ref, sum_ref, max_ref, cnt_ref):
    i = pl.program_id(0)
    nsteps = pl.num_programs(0)

    @pl.when(i == 0)
    def _init():
        sum_ref[...] = jnp.zeros_like(sum_ref)
        cnt_ref[...] = jnp.zeros_like(cnt_ref)
        max_ref[...] = jnp.full_like(max_ref, -jnp.inf)

    h = h_ref[...]
    bt = b_ref[0]
    nb = sum_ref.shape[0]
    rows = h.shape[0]
    oh = (jax.lax.broadcasted_iota(jnp.int32, (rows, nb), 1)
          == bt).astype(jnp.float32)
    dn = (((0,), (0,)), ((), ()))
    sum_ref[...] += jax.lax.dot_general(
        oh, h, dn, preferred_element_type=jnp.float32)
    cnt_ref[...] += jax.lax.dot_general(
        oh, jnp.ones((rows, 1), jnp.float32), dn,
        preferred_element_type=jnp.float32)

    g0 = jnp.min(bt)
    g1 = jnp.max(bt)

    def _mx(g, _):
        mask = bt == g
        mx = jnp.max(jnp.where(mask, h, -jnp.inf), axis=0, keepdims=True)
        max_ref[pl.ds(g, 1), :] = jnp.maximum(max_ref[pl.ds(g, 1), :], mx)
        return 0

    jax.lax.fori_loop(g0, g1 + 1, _mx, 0)

    @pl.when(i == nsteps - 1)
    def _head():
        cnt = jnp.maximum(cnt_ref[...], 1.0)
        gmean = sum_ref[...] / cnt
        gmax = max_ref[...]
        gf = jnp.dot(gf_ref[...], gw1_ref[...],
                     preferred_element_type=jnp.float32) + gb1_ref[...]
        gf = jnp.maximum(_ln_rows(gf, gg_ref[...], gbl_ref[...]), 0.0)
        gf = jnp.maximum(
            jnp.dot(gf, gw2_ref[...], preferred_element_type=jnp.float32)
            + gb2_ref[...], 0.0)
        z = (jnp.dot(gmean, cw1_ref[0:H, :], preferred_element_type=jnp.float32)
             + jnp.dot(gmax, cw1_ref[H:2 * H, :], preferred_element_type=jnp.float32)
             + jnp.dot(gf, cw1_ref[2 * H:, :], preferred_element_type=jnp.float32))
        z = jnp.maximum(z + cb1_ref[...], 0.0)
        z = jnp.maximum(
            jnp.dot(z, cw2_ref[...], preferred_element_type=jnp.float32)
            + cb2_ref[...], 0.0)
        o_ref[...] = (jnp.dot(z, cw3_ref[...], preferred_element_type=jnp.float32)
                      + cb3_ref[...])


def _pool_head(h, batch, gene_feat, gp_W1, gp_b1, gp_g, gp_bln, gp_W2, gp_b2,
               cls_W1, cls_b1, cls_W2, cls_b2, cls_W3, cls_b3):
    n, _ = h.shape
    nb = gene_feat.shape[0]
    gf_dim = gene_feat.shape[1]
    grid = n // ROWS
    batch3 = batch.reshape(grid, ROWS, 1)
    gfp = jnp.pad(gene_feat, ((0, 0), (0, 128 - gf_dim)))
    gw1p = jnp.pad(gp_W1, ((0, 128 - gf_dim), (0, 0)))
    cw3p = jnp.pad(cls_W3, ((0, 0), (0, 128 - cls_W3.shape[1])))
    cb3p = jnp.pad(cls_b3.reshape(1, -1), ((0, 0), (0, 128 - cls_b3.shape[0])))

    out = pl.pallas_call(
        _pool_body,
        grid=(grid,),
        in_specs=[
            pl.BlockSpec((ROWS, H), lambda i: (i, 0)),
            pl.BlockSpec((1, ROWS, 1), lambda i: (i, 0, 0)),
            pl.BlockSpec((nb, 128), lambda i: (0, 0)),
            pl.BlockSpec((128, H), lambda i: (0, 0)),
            pl.BlockSpec((1, H), lambda i: (0, 0)),
            pl.BlockSpec((1, H), lambda i: (0, 0)),
            pl.BlockSpec((1, H), lambda i: (0, 0)),
            pl.BlockSpec((H, H // 2), lambda i: (0, 0)),
            pl.BlockSpec((1, H // 2), lambda i: (0, 0)),
            pl.BlockSpec((2 * H + H // 2, H), lambda i: (0, 0)),
            pl.BlockSpec((1, H), lambda i: (0, 0)),
            pl.BlockSpec((H, H // 2), lambda i: (0, 0)),
            pl.BlockSpec((1, H // 2), lambda i: (0, 0)),
            pl.BlockSpec((H // 2, 128), lambda i: (0, 0)),
            pl.BlockSpec((1, 128), lambda i: (0, 0)),
        ],
        out_specs=pl.BlockSpec((nb, 128), lambda i: (0, 0)),
        out_shape=jax.ShapeDtypeStruct((nb, 128), jnp.float32),
        scratch_shapes=[
            pltpu.VMEM((nb, H), jnp.float32),
            pltpu.VMEM((nb, H), jnp.float32),
            pltpu.VMEM((nb, 1), jnp.float32),
        ],
    )(h, batch3, gfp, gw1p, gp_b1.reshape(1, H), gp_g.reshape(1, H),
      gp_bln.reshape(1, H), gp_W2, gp_b2.reshape(1, H // 2),
      cls_W1, cls_b1.reshape(1, H), cls_W2, cls_b2.reshape(1, H // 2),
      cw3p, cb3p)
    return out[:, :cls_W3.shape[1]]


def _edge_agg(h, src, dst, ew):
    msg = ew[:, None] * h[src]
    return jnp.zeros_like(h).at[dst].add(msg)


def kernel(x, edge_index, edge_attr, batch, gene_feat, in_W, in_b, in_g,
           in_bln, gin_W1, gin_b1, gin_W2, gin_b2, eps, ln_g, ln_b, gp_W1,
           gp_b1, gp_g, gp_bln, gp_W2, gp_b2, cls_W1, cls_b1, cls_W2,
           cls_b2, cls_W3, cls_b3):
    h = _input_layer(x, in_W, in_b, in_g, in_bln)
    src = edge_index[0]
    dst = edge_index[1]
    ew = edge_attr.reshape(-1)
    L = gin_W1.shape[0]
    for i in range(L):
        agg = _edge_agg(h, src, dst, ew)
        h = _gin_dense(h, agg, 2.0 + eps[i], gin_W1[i], gin_b1[i],
                       gin_W2[i], gin_b2[i], ln_g[i], ln_b[i])
    return _pool_head(h, batch, gene_feat, gp_W1, gp_b1, gp_g, gp_bln,
                      gp_W2, gp_b2, cls_W1, cls_b1, cls_W2, cls_b2,
                      cls_W3, cls_b3)
